# Initial kernel scaffold; baseline (speedup 1.0000x reference)
import jax
import jax.numpy as jnp
from jax.experimental import pallas as pl


def kernel(input, target, label):
    raise NotImplementedError("write your pallas kernel here")


# speedup vs baseline: 7.0167x; 7.0167x over previous
import functools

import jax
import jax.numpy as jnp
from jax import lax
from jax.experimental import pallas as pl
from jax.experimental.pallas import tpu as pltpu
from jax.experimental.pallas import tpu_sc as plsc

_RATIOS = jnp.array(
    [0.317, 0.329, 0.326, 0.115, 0.701, 0.367, 1.22, 0.241], dtype=jnp.float32
)
_SMOOTH = 0.0001

_B = 32
_N = 32768
_L = 16
_NV = _N // _L
_MSB = jnp.uint32(0x80000000)
_SENT = jnp.uint32(0xFFFFFFFF)


def _sc_body(inp_hbm, tgt_hbm, ratio_hbm, thr_hbm,
             inp_v, tgt_v, ratio_v, keys_v, hist_v, out_v):
    wid = lax.axis_index("s") * 2 + lax.axis_index("c")
    pltpu.sync_copy(inp_hbm.at[wid], inp_v)
    pltpu.sync_copy(tgt_hbm.at[wid], tgt_v)
    pltpu.sync_copy(ratio_hbm.at[wid], ratio_v)

    def key_body(i, pos_acc):
        off = pl.multiple_of(i * _L, _L)
        x = inp_v[pl.ds(off, _L)]
        t = tgt_v[pl.ds(off, _L)]
        bits = plsc.bitcast(x, jnp.uint32)
        key = jnp.where((bits >> jnp.uint32(31)) == jnp.uint32(0),
                        bits | _MSB, ~bits)
        key = jnp.where(t > 0, _SENT, key)
        keys_v[pl.ds(off, _L)] = key
        return pos_acc + t

    pos_acc = lax.fori_loop(0, _NV, key_body, jnp.zeros((_L,), jnp.int32))
    pos_num = jnp.sum(pos_acc)
    neg_num = _N - pos_num
    ratio = jnp.max(ratio_v[...])
    keep = jnp.minimum((pos_num.astype(jnp.float32) * ratio).astype(jnp.int32),
                       neg_num)
    k_idx = jnp.where(keep > 1, neg_num - keep + 1, 1 - keep)

    ones = jnp.ones((_L,), jnp.int32)

    def level(prefix, prefix_len, digit_bits, k_rem):
        nbins = 1 << digit_bits
        shift = jnp.uint32(32 - prefix_len - digit_bits)

        def zero_body(i, _):
            hist_v[pl.ds(pl.multiple_of(i * _L, _L), _L)] = jnp.zeros(
                (_L,), jnp.int32)
            return 0

        lax.fori_loop(0, nbins // _L, zero_body, 0)

        def hist_body(i, _):
            k = keys_v[pl.ds(pl.multiple_of(i * _L, _L), _L)]
            b = ((k >> shift) & jnp.uint32(nbins - 1)).astype(jnp.int32)
            if prefix_len == 0:
                plsc.addupdate_scatter(hist_v, [b], ones)
            else:
                m = (k >> jnp.uint32(32 - prefix_len)) == prefix
                plsc.addupdate_scatter(hist_v, [b], ones, mask=m)
            return 0

        lax.fori_loop(0, _NV, hist_body, 0)

        def scan_body(i, carry):
            nbefore, cbefore, run = carry
            h = hist_v[pl.ds(pl.multiple_of(i * _L, _L), _L)]
            cum = jnp.cumsum(h) + run
            mle = cum <= k_rem
            nbefore = nbefore + jnp.max(plsc.all_reduce_population_count(mle))
            cbefore = jnp.maximum(cbefore, jnp.max(jnp.where(mle, cum, 0)))
            return nbefore, cbefore, jnp.max(cum)

        tbin, cbefore, _ = lax.fori_loop(
            0, nbins // _L, scan_body,
            (jnp.int32(0), jnp.int32(0), jnp.int32(0)))
        new_prefix = (prefix << jnp.uint32(digit_bits)) | tbin.astype(jnp.uint32)
        return new_prefix, k_rem - cbefore

    p, k = level(jnp.uint32(0), 0, 12, k_idx)
    p, k = level(p, 12, 12, k)
    p, _ = level(p, 24, 8, k)

    selv = jnp.broadcast_to(p, (_L,))
    bits = jnp.where((selv >> jnp.uint32(31)) == jnp.uint32(1),
                     selv ^ _MSB, ~selv)
    out_v[...] = plsc.bitcast(bits, jnp.float32)
    pltpu.sync_copy(out_v, thr_hbm.at[wid])


def _sc_threshold(inp, tgt, ratio16):
    call = functools.partial(
        pl.kernel,
        out_type=jax.ShapeDtypeStruct((_B, _L), jnp.float32),
        mesh=plsc.VectorSubcoreMesh(core_axis_name="c", subcore_axis_name="s"),
        compiler_params=pltpu.CompilerParams(needs_layout_passes=False),
        scratch_types=[
            pltpu.VMEM((_N,), jnp.float32),
            pltpu.VMEM((_N,), jnp.int32),
            pltpu.VMEM((_L,), jnp.float32),
            pltpu.VMEM((_N,), jnp.uint32),
            pltpu.VMEM((4096,), jnp.int32),
            pltpu.VMEM((_L,), jnp.float32),
        ],
    )(_sc_body)
    return call(inp, tgt, ratio16)


def _tc_body(x_ref, t_ref, thr_ref, out_ref):
    x = x_ref[...]
    t = t_ref[...]
    thr = thr_ref[...][:, 0:1]
    pos = t > 0
    s = 1.0 / (1.0 + jnp.exp(-x))
    keepm = (x > thr) | pos
    p = jnp.where(keepm, s, 0.0)
    q = 1.0 - p
    fi = q * q * p
    tf = t.astype(jnp.float32)
    inter = jnp.sum(fi * tf, axis=-1, keepdims=True)
    denom = jnp.sum(fi) + jnp.sum(tf)
    out_ref[...] = 1.0 - (2.0 * inter + _SMOOTH) / (denom + _SMOOTH)


def kernel(input, target, label):
    ratio = _RATIOS[label]
    ratio16 = jnp.broadcast_to(ratio[:, None], (_B, _L))
    thr = _sc_threshold(input, target, ratio16)
    loss = pl.pallas_call(
        _tc_body,
        out_shape=jax.ShapeDtypeStruct((_B, 1), jnp.float32),
    )(input, target, thr)
    return loss[:, 0]


# speedup vs baseline: 15.0039x; 2.1383x over previous
import functools

import jax
import jax.numpy as jnp
from jax import lax
from jax.experimental import pallas as pl
from jax.experimental.pallas import tpu as pltpu
from jax.experimental.pallas import tpu_sc as plsc

_RATIOS = jnp.array(
    [0.317, 0.329, 0.326, 0.115, 0.701, 0.367, 1.22, 0.241], dtype=jnp.float32
)
_SMOOTH = 0.0001

_B = 32
_N = 32768
_L = 16
_NV = _N // _L


_IMIN = jnp.int32(-2147483648)
_NB1 = 4096
_NB2 = 1024
_NB3 = 1024
_HWORDS = _NB1 + _NB2 + _NB3


def _sc_body(inp_hbm, tgt_hbm, ratio_hbm, thr_hbm,
             inp_v, tgt_v, ratio_v, keys_v, hist_v, out_v,
             sem1, sem2, sem3):
    wid = lax.axis_index("s") * 2 + lax.axis_index("c")
    h1 = pltpu.async_copy(inp_hbm.at[wid], inp_v, sem1)
    h2 = pltpu.async_copy(tgt_hbm.at[wid], tgt_v, sem2)
    h3 = pltpu.async_copy(ratio_hbm.at[wid], ratio_v, sem3)

    zeros = jnp.zeros((_L,), jnp.int32)
    ones = jnp.ones((_L,), jnp.int32)

    @plsc.parallel_loop(0, _HWORDS // _L, unroll=8)
    def _(i):
        hist_v[pl.ds(pl.multiple_of(i * _L, _L), _L)] = zeros

    h1.wait()
    h2.wait()
    h3.wait()

    @plsc.parallel_loop(0, _NV, unroll=8, carry=zeros)
    def pos_acc(i, acc):
        off = pl.multiple_of(i * _L, _L)
        x = inp_v[pl.ds(off, _L)]
        t = tgt_v[pl.ds(off, _L)]
        b = plsc.bitcast(x, jnp.int32)
        sgn = lax.shift_right_arithmetic(b, jnp.int32(31))
        key = b ^ (sgn | _IMIN)
        key = jnp.where(t > 0, jnp.int32(-1), key)
        keys_v[pl.ds(off, _L)] = key
        d1 = lax.shift_right_logical(key, jnp.int32(20))
        plsc.addupdate_scatter(hist_v, [d1], ones)
        return acc + t

    pos_num = jnp.sum(pos_acc)
    neg_num = _N - pos_num
    ratio = jnp.max(ratio_v[...])
    keep = jnp.minimum((pos_num.astype(jnp.float32) * ratio).astype(jnp.int32),
                       neg_num)
    k_idx = jnp.where(keep > 1, neg_num - keep + 1, 1 - keep)

    def scan_hist(base, nbins, k_rem):
        def body(i, carry):
            nb, cb, run = carry
            h = hist_v[pl.ds(base + pl.multiple_of(i * _L, _L), _L)]
            cum = jnp.cumsum(h) + run
            mle = cum <= k_rem
            nb = nb + jnp.where(mle, 1, 0)
            cb = cb + jnp.where(mle, h, 0)
            run = run + jnp.broadcast_to(jnp.sum(h), (_L,))
            return nb, cb, run

        nb, cb, _ = lax.fori_loop(0, nbins // _L, body, (zeros, zeros, zeros))
        return jnp.sum(nb), jnp.sum(cb)

    t1, cb1 = scan_hist(0, _NB1, k_idx)
    k2 = k_idx - cb1

    @plsc.parallel_loop(0, _NV, unroll=8)
    def _(i):
        k = keys_v[pl.ds(pl.multiple_of(i * _L, _L), _L)]
        m = lax.shift_right_logical(k, jnp.int32(20)) == t1
        d = lax.shift_right_logical(k, jnp.int32(10)) & jnp.int32(_NB2 - 1)
        plsc.addupdate_scatter(hist_v, [d + jnp.int32(_NB1)], ones, mask=m)

    t2, cb2 = scan_hist(_NB1, _NB2, k2)
    k3 = k2 - cb2
    p2 = (t1 << 10) | t2

    @plsc.parallel_loop(0, _NV, unroll=8)
    def _(i):
        k = keys_v[pl.ds(pl.multiple_of(i * _L, _L), _L)]
        m = lax.shift_right_logical(k, jnp.int32(10)) == p2
        d = k & jnp.int32(_NB3 - 1)
        plsc.addupdate_scatter(hist_v, [d + jnp.int32(_NB1 + _NB2)], ones,
                               mask=m)

    t3, _ = scan_hist(_NB1 + _NB2, _NB3, k3)
    sel = (p2 << 10) | t3

    selv = jnp.broadcast_to(sel, (_L,))
    bits = jnp.where(selv < 0, selv ^ _IMIN, ~selv)
    out_v[...] = plsc.bitcast(bits, jnp.float32)
    pltpu.sync_copy(out_v, thr_hbm.at[wid])


def _sc_threshold(inp, tgt, ratio16):
    call = functools.partial(
        pl.kernel,
        out_type=jax.ShapeDtypeStruct((_B, _L), jnp.float32),
        mesh=plsc.VectorSubcoreMesh(core_axis_name="c", subcore_axis_name="s"),
        compiler_params=pltpu.CompilerParams(needs_layout_passes=False),
        scratch_types=[
            pltpu.VMEM((_N,), jnp.float32),
            pltpu.VMEM((_N,), jnp.int32),
            pltpu.VMEM((_L,), jnp.float32),
            pltpu.VMEM((_N,), jnp.int32),
            pltpu.VMEM((_HWORDS,), jnp.int32),
            pltpu.VMEM((_L,), jnp.float32),
            pltpu.SemaphoreType.DMA,
            pltpu.SemaphoreType.DMA,
            pltpu.SemaphoreType.DMA,
        ],
    )(_sc_body)
    return call(inp, tgt, ratio16)


def _tc_body(x_ref, t_ref, thr_ref, out_ref):
    x = x_ref[...]
    t = t_ref[...]
    thr = thr_ref[...][:, 0:1]
    pos = t > 0
    s = 1.0 / (1.0 + jnp.exp(-x))
    keepm = (x > thr) | pos
    p = jnp.where(keepm, s, 0.0)
    q = 1.0 - p
    fi = q * q * p
    tf = t.astype(jnp.float32)
    inter = jnp.sum(fi * tf, axis=-1, keepdims=True)
    denom = jnp.sum(fi) + jnp.sum(tf)
    out_ref[...] = 1.0 - (2.0 * inter + _SMOOTH) / (denom + _SMOOTH)


def kernel(input, target, label):
    ratio = _RATIOS[label]
    ratio16 = jnp.broadcast_to(ratio[:, None], (_B, _L))
    thr = _sc_threshold(input, target, ratio16)
    loss = pl.pallas_call(
        _tc_body,
        out_shape=jax.ShapeDtypeStruct((_B, 1), jnp.float32),
    )(input, target, thr)
    return loss[:, 0]
